# baseline (speedup 1.0000x reference)
import functools

import jax
import jax.numpy as jnp
from jax import lax
from jax.experimental import pallas as pl
from jax.experimental.pallas import tpu as pltpu

_W = 32


def _tc_body(idxt_ref, x4_ref, tp_ref, loss_ref, mpl_ref, dl_ref,
             acc_ref, cnt_ref, mpl_acc, *, n_cells, blk4):
    i = pl.program_id(0)
    nblk = pl.num_programs(0)

    @pl.when(i == 0)
    def _init():
        acc_ref[...] = jnp.zeros_like(acc_ref)
        cnt_ref[...] = jnp.zeros_like(cnt_ref)
        mpl_acc[0] = 0.0

    x4 = x4_ref[...]
    idxt = idxt_ref[0]

    m4 = jnp.max(x4.reshape(blk4, 4, 32), axis=2)
    mpl_acc[0] += jnp.sum(jnp.log(m4))

    xb = x4.astype(jnp.bfloat16)
    first = jnp.min(idxt)
    last = jnp.max(idxt)
    base0 = (first // _W) * _W
    nwin = (last - base0) // _W + 1
    iota_w = lax.broadcasted_iota(jnp.int32, (_W, 1), 0).astype(jnp.bfloat16)
    onesb = jnp.ones((blk4, 8), dtype=jnp.bfloat16)

    def body(k, _):
        base = base0 + k * _W
        relb = (idxt - base).astype(jnp.bfloat16)
        oh = jnp.concatenate(
            [(iota_w == relb[j][None, :]).astype(jnp.bfloat16) for j in range(4)],
            axis=0)
        r = jnp.dot(oh, xb, preferred_element_type=jnp.float32)
        c = jnp.dot(oh, onesb, preferred_element_type=jnp.float32)
        s = (r[0:_W, 0:32] + r[_W:2 * _W, 32:64]
             + r[2 * _W:3 * _W, 64:96] + r[3 * _W:4 * _W, 96:128])
        cw = c[0:_W] + c[_W:2 * _W] + c[2 * _W:3 * _W] + c[3 * _W:4 * _W]
        acc_ref[pl.ds(base, _W), :] += s
        cnt_ref[pl.ds(base, _W), :] += cw
        return 0

    lax.fori_loop(0, nwin, body, 0)

    @pl.when(i == nblk - 1)
    def _fin():
        sums = acc_ref[...]
        cnts = cnt_ref[...][:, 0:1]
        pred = sums / jnp.maximum(cnts, 1.0)
        dl = jnp.mean(jnp.abs(pred - tp_ref[...]))
        mpl = -mpl_acc[0] / n_cells
        loss = 0.5 * mpl + 0.5 * dl
        loss_ref[...] = jnp.full((1, 1), loss, jnp.float32)
        mpl_ref[...] = jnp.full((1, 1), mpl, jnp.float32)
        dl_ref[...] = jnp.full((1, 1), dl, jnp.float32)


def _pick_block(n4):
    for b in (2000, 1000, 500, 250, 200, 100, 50, 25, 20, 10, 5, 4, 2, 1):
        if n4 % b == 0 and b <= n4:
            return b
    return n4


def kernel(outputs, bag_indices, true_proportions):
    n_cells, n_classes = outputs.shape
    n_bags = true_proportions.shape[0]
    assert n_classes == 32 and n_cells % 4 == 0
    n4 = n_cells // 4
    blk4 = _pick_block(n4)
    nblk = n4 // blk4

    x4 = outputs.reshape(n4, 128)
    idxt = bag_indices.astype(jnp.int32).reshape(nblk, blk4, 4).transpose(0, 2, 1)

    body = functools.partial(_tc_body, n_cells=n_cells, blk4=blk4)

    out = pl.pallas_call(
        body,
        grid=(nblk,),
        in_specs=[
            pl.BlockSpec((1, 4, blk4), lambda i: (i, 0, 0)),
            pl.BlockSpec((blk4, 128), lambda i: (i, 0)),
            pl.BlockSpec((n_bags, n_classes), lambda i: (0, 0)),
        ],
        out_specs=[
            pl.BlockSpec((1, 1), lambda i: (0, 0)),
            pl.BlockSpec((1, 1), lambda i: (0, 0)),
            pl.BlockSpec((1, 1), lambda i: (0, 0)),
        ],
        out_shape=[
            jax.ShapeDtypeStruct((1, 1), jnp.float32),
            jax.ShapeDtypeStruct((1, 1), jnp.float32),
            jax.ShapeDtypeStruct((1, 1), jnp.float32),
        ],
        scratch_shapes=[
            pltpu.VMEM((n_bags, 32), jnp.float32),
            pltpu.VMEM((n_bags, 8), jnp.float32),
            pltpu.SMEM((1,), jnp.float32),
        ],
    )(idxt, x4, true_proportions)

    loss, mpl, dl = out
    return (loss[0, 0], mpl[0, 0], dl[0, 0])


# speedup vs baseline: 1.9886x; 1.9886x over previous
import functools

import jax
import jax.numpy as jnp
from jax import lax
from jax.experimental import pallas as pl
from jax.experimental.pallas import tpu as pltpu

_W = 32


def _tc_body(idx_ref, x_ref, tp_ref, loss_ref, mpl_ref, dl_ref,
             acc_ref, mpl_acc, *, n_cells, blk):
    i = pl.program_id(0)
    nblk = pl.num_programs(0)

    @pl.when(i == 0)
    def _init():
        acc_ref[...] = jnp.zeros_like(acc_ref)
        mpl_acc[0] = 0.0

    x = x_ref[...]
    idx = idx_ref[0, 0, :]

    m = jnp.max(x, axis=1)
    lg = jnp.log(m)[:, None]
    ones_row = jnp.ones((1, blk), jnp.float32)
    mpl_acc[0] += jnp.dot(ones_row, lg, preferred_element_type=jnp.float32)[0, 0]

    xb = x.astype(jnp.bfloat16)
    rhs = jnp.concatenate(
        [xb, jnp.ones((blk, 1), jnp.bfloat16), jnp.zeros((blk, 31), jnp.bfloat16)],
        axis=1)

    first = jnp.min(idx)
    last = jnp.max(idx)
    base0 = (first // _W) * _W
    nwin = (last - base0) // _W + 1
    iota_w = lax.broadcasted_iota(jnp.int32, (_W, 1), 0).astype(jnp.bfloat16)

    def body(k, _):
        base = base0 + k * _W
        relb = (idx - base).astype(jnp.bfloat16)
        ohT = (iota_w == relb[None, :]).astype(jnp.bfloat16)
        win = jnp.dot(ohT, rhs, preferred_element_type=jnp.float32)
        acc_ref[pl.ds(base, _W), :] += win
        return 0

    lax.fori_loop(0, nwin, body, 0)

    @pl.when(i == nblk - 1)
    def _fin():
        acc = acc_ref[...]
        sums = acc[:, :32]
        cnts = acc[:, 32:33]
        pred = sums / jnp.maximum(cnts, 1.0)
        dl = jnp.mean(jnp.abs(pred - tp_ref[...]))
        mpl = -mpl_acc[0] / n_cells
        loss = 0.5 * mpl + 0.5 * dl
        loss_ref[...] = jnp.full((1, 1), loss, jnp.float32)
        mpl_ref[...] = jnp.full((1, 1), mpl, jnp.float32)
        dl_ref[...] = jnp.full((1, 1), dl, jnp.float32)


def _pick_block(n):
    for b in (8000, 4000, 2000, 1000, 500, 200, 100, 50, 20, 10):
        if n % b == 0 and b % 8 == 0 and b <= n:
            return b
    return n


def kernel(outputs, bag_indices, true_proportions):
    n_cells, n_classes = outputs.shape
    n_bags = true_proportions.shape[0]
    blk = _pick_block(n_cells)
    nblk = n_cells // blk
    idx3 = bag_indices.astype(jnp.int32).reshape(nblk, 1, blk)

    body = functools.partial(_tc_body, n_cells=n_cells, blk=blk)

    out = pl.pallas_call(
        body,
        grid=(nblk,),
        in_specs=[
            pl.BlockSpec((1, 1, blk), lambda i: (i, 0, 0)),
            pl.BlockSpec((blk, n_classes), lambda i: (i, 0)),
            pl.BlockSpec((n_bags, n_classes), lambda i: (0, 0)),
        ],
        out_specs=[
            pl.BlockSpec((1, 1), lambda i: (0, 0)),
            pl.BlockSpec((1, 1), lambda i: (0, 0)),
            pl.BlockSpec((1, 1), lambda i: (0, 0)),
        ],
        out_shape=[
            jax.ShapeDtypeStruct((1, 1), jnp.float32),
            jax.ShapeDtypeStruct((1, 1), jnp.float32),
            jax.ShapeDtypeStruct((1, 1), jnp.float32),
        ],
        scratch_shapes=[
            pltpu.VMEM((n_bags, 64), jnp.float32),
            pltpu.SMEM((1,), jnp.float32),
        ],
    )(idx3, outputs, true_proportions)

    loss, mpl, dl = out
    return (loss[0, 0], mpl[0, 0], dl[0, 0])


# speedup vs baseline: 1.9927x; 1.0020x over previous
import functools

import jax
import jax.numpy as jnp
from jax import lax
from jax.experimental import pallas as pl
from jax.experimental.pallas import tpu as pltpu

_W = 32


def _tc_body(idx_ref, x_ref, tp_ref, loss_ref, mpl_ref, dl_ref,
             acc_ref, mpl_acc, *, n_cells, blk):
    i = pl.program_id(0)
    nblk = pl.num_programs(0)

    @pl.when(i == 0)
    def _init():
        acc_ref[...] = jnp.zeros_like(acc_ref)
        mpl_acc[0] = 0.0

    x = x_ref[...]
    idx8 = idx_ref[...]

    m = jnp.max(x, axis=1)
    lg = jnp.log(m)[:, None]
    ones_row = jnp.ones((1, blk), jnp.float32)
    mpl_acc[0] += jnp.dot(ones_row, lg, preferred_element_type=jnp.float32)[0, 0]

    xb = x.astype(jnp.bfloat16)
    rhs = jnp.concatenate(
        [xb, jnp.ones((blk, 1), jnp.bfloat16), jnp.zeros((blk, 31), jnp.bfloat16)],
        axis=1)

    first = jnp.min(idx8)
    last = jnp.max(idx8)
    base0 = (first // _W) * _W
    nwin = (last - base0) // _W + 1
    iota_w = lax.broadcasted_iota(jnp.int32, (_W, 1), 0).astype(jnp.bfloat16)
    sub = blk // 8

    def body(k, _):
        base = base0 + k * _W
        relb = (idx8 - base).astype(jnp.bfloat16)
        win = jnp.zeros((_W, 64), jnp.float32)
        for j in range(8):
            ohT = (iota_w == relb[j][None, :]).astype(jnp.bfloat16)
            win = win + jnp.dot(ohT, rhs[j * sub:(j + 1) * sub],
                                preferred_element_type=jnp.float32)
        acc_ref[pl.ds(base, _W), :] += win
        return 0

    lax.fori_loop(0, nwin, body, 0)

    @pl.when(i == nblk - 1)
    def _fin():
        acc = acc_ref[...]
        sums = acc[:, :32]
        cnts = acc[:, 32:33]
        pred = sums / jnp.maximum(cnts, 1.0)
        dl = jnp.mean(jnp.abs(pred - tp_ref[...]))
        mpl = -mpl_acc[0] / n_cells
        loss = 0.5 * mpl + 0.5 * dl
        loss_ref[...] = jnp.full((1, 1), loss, jnp.float32)
        mpl_ref[...] = jnp.full((1, 1), mpl, jnp.float32)
        dl_ref[...] = jnp.full((1, 1), dl, jnp.float32)


def _pick_block(n):
    for b in (8000, 4000, 2000, 1000, 500, 200, 100, 50, 20, 10):
        if n % b == 0 and b % 8 == 0 and b <= n:
            return b
    return n


def kernel(outputs, bag_indices, true_proportions):
    n_cells, n_classes = outputs.shape
    n_bags = true_proportions.shape[0]
    blk = _pick_block(n_cells)
    nblk = n_cells // blk
    idx2 = bag_indices.astype(jnp.int32).reshape(nblk * 8, blk // 8)

    body = functools.partial(_tc_body, n_cells=n_cells, blk=blk)

    out = pl.pallas_call(
        body,
        grid=(nblk,),
        in_specs=[
            pl.BlockSpec((8, blk // 8), lambda i: (i, 0)),
            pl.BlockSpec((blk, n_classes), lambda i: (i, 0)),
            pl.BlockSpec((n_bags, n_classes), lambda i: (0, 0)),
        ],
        out_specs=[
            pl.BlockSpec((1, 1), lambda i: (0, 0)),
            pl.BlockSpec((1, 1), lambda i: (0, 0)),
            pl.BlockSpec((1, 1), lambda i: (0, 0)),
        ],
        out_shape=[
            jax.ShapeDtypeStruct((1, 1), jnp.float32),
            jax.ShapeDtypeStruct((1, 1), jnp.float32),
            jax.ShapeDtypeStruct((1, 1), jnp.float32),
        ],
        scratch_shapes=[
            pltpu.VMEM((n_bags, 64), jnp.float32),
            pltpu.SMEM((1,), jnp.float32),
        ],
    )(idx2, outputs, true_proportions)

    loss, mpl, dl = out
    return (loss[0, 0], mpl[0, 0], dl[0, 0])


# speedup vs baseline: 7.3198x; 3.6733x over previous
import functools

import jax
import jax.numpy as jnp
from jax import lax
from jax.experimental import pallas as pl
from jax.experimental.pallas import tpu as pltpu

_W = 32
_B = 8192


def _tc_body(idx_ref, xt_ref, tp_ref, loss_ref, mpl_ref, dl_ref,
             acc_ref, mpl_acc, *, n_cells, blk):
    i = pl.program_id(0)
    nblk = pl.num_programs(0)

    @pl.when(i == 0)
    def _init():
        acc_ref[...] = jnp.zeros_like(acc_ref)
        mpl_acc[0] = 0.0

    xt = xt_ref[...]
    idx = idx_ref[...]
    valid = n_cells - i * blk
    lmask = lax.broadcasted_iota(jnp.int32, (1, blk), 1) < valid

    m = jnp.max(xt, axis=0, keepdims=True)
    m = jnp.where(lmask, m, 1.0)
    mpl_acc[0] += jnp.sum(jnp.log(m))

    xtb = jnp.where(lmask, xt, 0.0).astype(jnp.bfloat16)
    aug = jnp.concatenate(
        [xtb,
         jnp.ones((1, blk), jnp.bfloat16),
         jnp.zeros((7, blk), jnp.bfloat16)], axis=0)

    idxm = jnp.where(lmask[0], idx, -2048)
    first = jnp.min(jnp.where(lmask[0], idx, 1023))
    last = jnp.max(idxm)
    base0 = (first // _W) * _W
    nwin = (last - base0) // _W + 1
    iota_w = lax.broadcasted_iota(jnp.int32, (_W, 1), 0).astype(jnp.bfloat16)

    def body(k, _):
        base = base0 + k * _W
        relb = (idxm - base).astype(jnp.bfloat16)
        oh = (iota_w == relb[None, :]).astype(jnp.bfloat16)
        win = lax.dot_general(aug, oh, (((1,), (1,)), ((), ())),
                              preferred_element_type=jnp.float32)
        acc_ref[pl.ds(base, _W), :] += win.T
        return 0

    lax.fori_loop(0, nwin, body, 0)

    @pl.when(i == nblk - 1)
    def _fin():
        acc = acc_ref[...]
        sums = acc[:, :32]
        cnts = acc[:, 32:33]
        pred = sums / jnp.maximum(cnts, 1.0)
        dl = jnp.mean(jnp.abs(pred - tp_ref[...]))
        mpl = -mpl_acc[0] / n_cells
        loss = 0.5 * mpl + 0.5 * dl
        loss_ref[...] = jnp.full((1, 1), loss, jnp.float32)
        mpl_ref[...] = jnp.full((1, 1), mpl, jnp.float32)
        dl_ref[...] = jnp.full((1, 1), dl, jnp.float32)


def kernel(outputs, bag_indices, true_proportions):
    n_cells, n_classes = outputs.shape
    n_bags = true_proportions.shape[0]
    blk = _B
    nblk = (n_cells + blk - 1) // blk

    xt = outputs.T
    idx1 = bag_indices.astype(jnp.int32)

    body = functools.partial(_tc_body, n_cells=n_cells, blk=blk)

    out = pl.pallas_call(
        body,
        grid=(nblk,),
        in_specs=[
            pl.BlockSpec((blk,), lambda i: (i,)),
            pl.BlockSpec((n_classes, blk), lambda i: (0, i)),
            pl.BlockSpec((n_bags, n_classes), lambda i: (0, 0)),
        ],
        out_specs=[
            pl.BlockSpec((1, 1), lambda i: (0, 0)),
            pl.BlockSpec((1, 1), lambda i: (0, 0)),
            pl.BlockSpec((1, 1), lambda i: (0, 0)),
        ],
        out_shape=[
            jax.ShapeDtypeStruct((1, 1), jnp.float32),
            jax.ShapeDtypeStruct((1, 1), jnp.float32),
            jax.ShapeDtypeStruct((1, 1), jnp.float32),
        ],
        scratch_shapes=[
            pltpu.VMEM((n_bags, 40), jnp.float32),
            pltpu.SMEM((1,), jnp.float32),
        ],
    )(idx1, xt, true_proportions)

    loss, mpl, dl = out
    return (loss[0, 0], mpl[0, 0], dl[0, 0])


# speedup vs baseline: 7.4237x; 1.0142x over previous
import functools

import jax
import jax.numpy as jnp
from jax import lax
from jax.experimental import pallas as pl
from jax.experimental.pallas import tpu as pltpu

_W = 32
_B = 8192


def _main_body(b0_ref, nw_ref, idx_ref, xt_ref, acc_ref, mplsum_ref,
               mpl_acc, *, n_cells, blk):
    i = pl.program_id(0)
    nblk = pl.num_programs(0)

    @pl.when(i == 0)
    def _init():
        acc_ref[...] = jnp.zeros_like(acc_ref)
        mpl_acc[0] = 0.0

    xt = xt_ref[...]
    idx = idx_ref[...]
    valid = n_cells - i * blk
    lmask = lax.broadcasted_iota(jnp.int32, (1, blk), 1) < valid

    m = jnp.max(xt, axis=0, keepdims=True)
    m = jnp.where(lmask, m, 1.0)
    mpl_acc[0] += jnp.sum(jnp.log(m))

    xtb = jnp.where(lmask, xt.astype(jnp.bfloat16), jnp.bfloat16(0))
    aug = jnp.concatenate(
        [xtb,
         jnp.ones((1, blk), jnp.bfloat16),
         jnp.zeros((7, blk), jnp.bfloat16)], axis=0)

    idxm = jnp.where(lmask[0], idx, -2048)
    base0 = b0_ref[i]
    nwin = nw_ref[i]
    iota_w = lax.broadcasted_iota(jnp.int32, (_W, 1), 0).astype(jnp.bfloat16)

    def body(k, _):
        base = base0 + k * _W
        relb = (idxm - base).astype(jnp.bfloat16)
        oh = (iota_w == relb[None, :]).astype(jnp.bfloat16)
        win = lax.dot_general(aug, oh, (((1,), (1,)), ((), ())),
                              preferred_element_type=jnp.float32)
        acc_ref[pl.ds(base, _W), :] += win.T
        return 0

    lax.fori_loop(0, nwin, body, 0)

    @pl.when(i == nblk - 1)
    def _fin():
        mplsum_ref[...] = jnp.full((1, 1), mpl_acc[0], jnp.float32)


def _fin_body(acc_ref, tp_ref, mplsum_ref, loss_ref, mpl_ref, dl_ref,
              *, n_cells):
    acc = acc_ref[...]
    sums = acc[:, :32]
    cnts = acc[:, 32:33]
    pred = sums / jnp.maximum(cnts, 1.0)
    dl = jnp.mean(jnp.abs(pred - tp_ref[...]))
    mpl = -jnp.sum(mplsum_ref[...]) / n_cells
    loss = 0.5 * mpl + 0.5 * dl
    loss_ref[...] = jnp.full((1, 1), loss, jnp.float32)
    mpl_ref[...] = jnp.full((1, 1), mpl, jnp.float32)
    dl_ref[...] = jnp.full((1, 1), dl, jnp.float32)


def kernel(outputs, bag_indices, true_proportions):
    n_cells, n_classes = outputs.shape
    n_bags = true_proportions.shape[0]
    blk = _B
    nblk = (n_cells + blk - 1) // blk

    xt = outputs.T
    idx1 = bag_indices.astype(jnp.int32)

    starts = jnp.arange(nblk, dtype=jnp.int32) * blk
    firsts = idx1[starts]
    lasts = idx1[jnp.minimum(starts + blk - 1, n_cells - 1)]
    base0s = (firsts // _W) * _W
    nwins = (lasts - base0s) // _W + 1

    main = functools.partial(_main_body, n_cells=n_cells, blk=blk)
    acc, mplsum = pl.pallas_call(
        main,
        grid=(nblk,),
        in_specs=[
            pl.BlockSpec(memory_space=pltpu.SMEM),
            pl.BlockSpec(memory_space=pltpu.SMEM),
            pl.BlockSpec((blk,), lambda i: (i,)),
            pl.BlockSpec((n_classes, blk), lambda i: (0, i)),
        ],
        out_specs=[
            pl.BlockSpec((n_bags, 40), lambda i: (0, 0)),
            pl.BlockSpec((1, 1), lambda i: (0, 0)),
        ],
        out_shape=[
            jax.ShapeDtypeStruct((n_bags, 40), jnp.float32),
            jax.ShapeDtypeStruct((1, 1), jnp.float32),
        ],
        scratch_shapes=[
            pltpu.SMEM((1,), jnp.float32),
        ],
    )(base0s, nwins, idx1, xt)

    fin = functools.partial(_fin_body, n_cells=n_cells)
    loss, mpl, dl = pl.pallas_call(
        fin,
        out_shape=[
            jax.ShapeDtypeStruct((1, 1), jnp.float32),
            jax.ShapeDtypeStruct((1, 1), jnp.float32),
            jax.ShapeDtypeStruct((1, 1), jnp.float32),
        ],
    )(acc, true_proportions, mplsum)

    return (loss[0, 0], mpl[0, 0], dl[0, 0])


# speedup vs baseline: 8.0834x; 1.0889x over previous
import functools

import jax
import jax.numpy as jnp
from jax import lax
from jax.experimental import pallas as pl
from jax.experimental.pallas import tpu as pltpu

_W = 32
_B = 8192


def _main_body(b0_ref, nw_ref, idxf_ref, xt_ref, acc_ref, mplv_ref, *, blk):
    i = pl.program_id(0)

    @pl.when(i == 0)
    def _init():
        acc_ref[...] = jnp.zeros_like(acc_ref)
        mplv_ref[...] = jnp.zeros_like(mplv_ref)

    xt = xt_ref[...]
    idxf = idxf_ref[...]

    lg = jnp.log(jnp.max(xt, axis=0, keepdims=True))
    part = lg[:, 0:1024]
    for j in range(1, blk // 1024):
        part = part + lg[:, j * 1024:(j + 1) * 1024]
    mplv_ref[...] += part

    aug = jnp.concatenate(
        [xt.astype(jnp.bfloat16),
         jnp.ones((1, blk), jnp.bfloat16),
         jnp.zeros((7, blk), jnp.bfloat16)], axis=0)

    base0 = b0_ref[i]
    nwin = nw_ref[i]
    iota_w = lax.broadcasted_iota(jnp.int32, (_W, 1), 0).astype(jnp.bfloat16)

    def body(k, _):
        base = base0 + k * _W
        relb = (idxf - base.astype(jnp.float32)).astype(jnp.bfloat16)
        oh = (iota_w == relb[None, :]).astype(jnp.bfloat16)
        win = lax.dot_general(oh, aug, (((1,), (1,)), ((), ())),
                              preferred_element_type=jnp.float32)
        acc_ref[pl.ds(base, _W), :] += win
        return 0

    lax.fori_loop(0, nwin, body, 0)


def _fin_body(idxf_ref, xtt_ref, acc_ref, mplv_ref, tp_ref,
              loss_ref, mpl_ref, dl_ref, *, n_cells, tail, tb, n_bags):
    acc = acc_ref[...]
    mpl_sum = jnp.sum(mplv_ref[...])

    if tail:
        lmask = lax.broadcasted_iota(jnp.int32, (1, tb), 1) < tail
        xt = xtt_ref[...]
        m = jnp.where(lmask, jnp.max(xt, axis=0, keepdims=True), 1.0)
        mpl_sum = mpl_sum + jnp.sum(jnp.log(m))
        xtb = jnp.where(lmask, xt.astype(jnp.bfloat16), jnp.bfloat16(0))
        aug = jnp.concatenate(
            [xtb,
             jnp.ones((1, tb), jnp.bfloat16),
             jnp.zeros((7, tb), jnp.bfloat16)], axis=0)
        idxf = jnp.where(lmask[0], idxf_ref[...], -1.0)
        iota_b = lax.broadcasted_iota(jnp.int32, (n_bags, 1), 0).astype(jnp.float32)
        oh = (iota_b == idxf[None, :]).astype(jnp.bfloat16)
        acc = acc + lax.dot_general(oh, aug, (((1,), (1,)), ((), ())),
                                    preferred_element_type=jnp.float32)

    sums = acc[:, :32]
    cnts = acc[:, 32:33]
    pred = sums / jnp.maximum(cnts, 1.0)
    dl = jnp.mean(jnp.abs(pred - tp_ref[...]))
    mpl = -mpl_sum / n_cells
    loss = 0.5 * mpl + 0.5 * dl
    loss_ref[...] = jnp.full((1, 1), loss, jnp.float32)
    mpl_ref[...] = jnp.full((1, 1), mpl, jnp.float32)
    dl_ref[...] = jnp.full((1, 1), dl, jnp.float32)


def kernel(outputs, bag_indices, true_proportions):
    n_cells, n_classes = outputs.shape
    n_bags = true_proportions.shape[0]
    blk = _B
    nfull = n_cells // blk
    tail = n_cells - nfull * blk

    xt = outputs.T
    idx1 = bag_indices.astype(jnp.int32)
    idxf = bag_indices.astype(jnp.float32)

    starts = jnp.arange(nfull, dtype=jnp.int32) * blk
    firsts = idx1[starts]
    lasts = idx1[starts + blk - 1]
    base0s = (firsts // _W) * _W
    nwins = (lasts - base0s) // _W + 1

    main = functools.partial(_main_body, blk=blk)
    acc, mplv = pl.pallas_call(
        main,
        grid=(nfull,),
        in_specs=[
            pl.BlockSpec(memory_space=pltpu.SMEM),
            pl.BlockSpec(memory_space=pltpu.SMEM),
            pl.BlockSpec((blk,), lambda i: (i,)),
            pl.BlockSpec((n_classes, blk), lambda i: (0, i)),
        ],
        out_specs=[
            pl.BlockSpec((n_bags, 40), lambda i: (0, 0)),
            pl.BlockSpec((1, 1024), lambda i: (0, 0)),
        ],
        out_shape=[
            jax.ShapeDtypeStruct((n_bags, 40), jnp.float32),
            jax.ShapeDtypeStruct((1, 1024), jnp.float32),
        ],
    )(base0s, nwins, idxf, xt)

    tb = 1024
    while tb < tail:
        tb *= 2
    tstart = nfull * blk // tb

    fin = functools.partial(_fin_body, n_cells=n_cells, tail=tail, tb=tb,
                            n_bags=n_bags)
    loss, mpl, dl = pl.pallas_call(
        fin,
        grid=(1,),
        in_specs=[
            pl.BlockSpec((tb,), lambda i: (tstart,)),
            pl.BlockSpec((n_classes, tb), lambda i: (0, tstart)),
            pl.BlockSpec((n_bags, 40), lambda i: (0, 0)),
            pl.BlockSpec((1, 1024), lambda i: (0, 0)),
            pl.BlockSpec((n_bags, n_classes), lambda i: (0, 0)),
        ],
        out_specs=[
            pl.BlockSpec((1, 1), lambda i: (0, 0)),
            pl.BlockSpec((1, 1), lambda i: (0, 0)),
            pl.BlockSpec((1, 1), lambda i: (0, 0)),
        ],
        out_shape=[
            jax.ShapeDtypeStruct((1, 1), jnp.float32),
            jax.ShapeDtypeStruct((1, 1), jnp.float32),
            jax.ShapeDtypeStruct((1, 1), jnp.float32),
        ],
    )(idxf, xt, acc, mplv, true_proportions)

    return (loss[0, 0], mpl[0, 0], dl[0, 0])
